# baseline (speedup 1.0000x reference)
import functools

import jax
import jax.numpy as jnp
from jax import lax
from jax.experimental import pallas as pl
from jax.experimental.pallas import tpu as pltpu
from jax.experimental.pallas import tpu_sc as plsc

N = 50000
E = 800000
EP = 819200
R = EP // 128
RPW = R // 32
K = 8
KB = 50
C0R = 150
C1R = 2 * RPW - C0R
C0TOT = 16 * C0R
NACC = 50048
ZSTR = NACC // 16
OSTR = N // 16
BN = 2000
BE = 8192

def _f32(*shape):
    return jax.ShapeDtypeStruct(shape, jnp.float32)


@functools.lru_cache(maxsize=None)
def _mesh():
    return plsc.VectorSubcoreMesh(core_axis_name="c", subcore_axis_name="s")


@functools.lru_cache(maxsize=None)
def _make_degree():
    return functools.partial(
        pl.kernel,
        out_type=_f32(2, NACC, 16),
        mesh=_mesh(),
        compiler_params=pltpu.CompilerParams(use_tc_tiling_on_sc=False),
        scratch_types=[
            pltpu.VMEM((K, 128), jnp.int32),
            pltpu.VMEM((128, 16), jnp.float32),
            pltpu.VMEM_SHARED((NACC, 16), jnp.float32),
        ],
    )(_sc_degree_body)


def _sc_degree_body(rows2d, zeros16, ones128, out, rowv, onesv, acc):
    c = lax.axis_index("c")
    s = lax.axis_index("s")
    pltpu.sync_copy(zeros16.at[pl.ds(s * ZSTR, ZSTR)], acc.at[pl.ds(s * ZSTR, ZSTR)])
    pltpu.sync_copy(ones128, onesv)
    plsc.subcore_barrier()
    rbase = (c * 16 + s) * RPW

    def body(i, _):
        pltpu.sync_copy(rows2d.at[pl.ds(rbase + i * K, K)], rowv)
        for j in range(K):
            pltpu.sync_copy(onesv, acc.at[rowv.at[j]], add=True)
        return 0

    lax.fori_loop(0, RPW // K, body, 0)
    plsc.subcore_barrier()
    pltpu.sync_copy(acc.at[pl.ds(s * ZSTR, ZSTR)], out.at[c].at[pl.ds(s * ZSTR, ZSTR)])


@functools.lru_cache(maxsize=None)
def _make_segsum(D, nchunk):
    def body(*refs):
        tables = refs[:nchunk]
        rowg2d = refs[nchunk]
        cols2d = refs[nchunk + 1]
        zeros = refs[nchunk + 2]
        outs = refs[nchunk + 3:nchunk + 3 + nchunk]
        rowv, colv, g0, g1, acc, sa0, sa1, c0, c1 = refs[nchunk + 3 + nchunk:]
        c = lax.axis_index("c")
        s = lax.axis_index("s")
        rbase = jnp.where(c == 0, s * C0R, C0TOT + s * C1R)
        nblk = jnp.where(c == 0, C0R // KB, C1R // KB)
        for p in range(nchunk):
            pltpu.sync_copy(zeros.at[pl.ds(s * ZSTR, ZSTR)], acc.at[pl.ds(s * ZSTR, ZSTR)])
            plsc.subcore_barrier()

            def blk(b, _):
                pltpu.sync_copy(rowg2d.at[pl.ds(rbase + b * KB, KB)], rowv)
                pltpu.sync_copy(cols2d.at[pl.ds(rbase + b * KB, KB)], colv)
                pltpu.async_copy(tables[p].at[rowv.at[0]], g0, sa0)

                def bd(i, _):
                    pltpu.make_async_copy(tables[p].at[rowv.at[2 * i]], g0, sa0).wait()
                    pltpu.async_copy(g0, acc.at[colv.at[2 * i]], c0, add=True)

                    @pl.when(i > 0)
                    def _():
                        pltpu.make_async_copy(g1, acc.at[colv.at[0]], c1).wait()

                    pltpu.async_copy(tables[p].at[rowv.at[2 * i + 1]], g1, sa1)
                    pltpu.make_async_copy(tables[p].at[rowv.at[2 * i + 1]], g1, sa1).wait()
                    pltpu.async_copy(g1, acc.at[colv.at[2 * i + 1]], c1, add=True)
                    pltpu.make_async_copy(g0, acc.at[colv.at[0]], c0).wait()
                    nxt = jnp.minimum(2 * i + 2, KB - 1)
                    pltpu.async_copy(tables[p].at[rowv.at[nxt]], g0, sa0)
                    return 0

                lax.fori_loop(0, KB // 2, bd, 0)
                pltpu.make_async_copy(g1, acc.at[colv.at[0]], c1).wait()
                pltpu.make_async_copy(tables[p].at[rowv.at[0]], g0, sa0).wait()
                return 0

            lax.fori_loop(0, nblk, blk, 0)
            plsc.subcore_barrier()
            pltpu.sync_copy(acc.at[pl.ds(s * ZSTR, ZSTR)],
                            outs[p].at[c].at[pl.ds(s * ZSTR, ZSTR)])
            plsc.subcore_barrier()

    return functools.partial(
        pl.kernel,
        out_type=[_f32(2, NACC, D)] * nchunk,
        mesh=_mesh(),
        compiler_params=pltpu.CompilerParams(use_tc_tiling_on_sc=False),
        scratch_types=[
            pltpu.VMEM((KB, 128), jnp.int32),
            pltpu.VMEM((KB, 128), jnp.int32),
            pltpu.VMEM((128, D), jnp.float32),
            pltpu.VMEM((128, D), jnp.float32),
            pltpu.VMEM_SHARED((NACC, D), jnp.float32),
            pltpu.SemaphoreType.DMA,
            pltpu.SemaphoreType.DMA,
            pltpu.SemaphoreType.DMA,
            pltpu.SemaphoreType.DMA,
        ],
    )(body)


@functools.lru_cache(maxsize=None)
def _make_gather_prod():
    return functools.partial(
        pl.kernel,
        out_type=[_f32(EP, 16), _f32(EP, 16)],
        mesh=_mesh(),
        compiler_params=pltpu.CompilerParams(use_tc_tiling_on_sc=False),
        scratch_types=[
            pltpu.VMEM((C1R, 128), jnp.int32),
            pltpu.VMEM((C1R, 128), jnp.int32),
            pltpu.VMEM((128, 112), jnp.float32),
            pltpu.VMEM((128, 112), jnp.float32),
            pltpu.VMEM((128, 112), jnp.float32),
            pltpu.VMEM((128, 112), jnp.float32),
            pltpu.VMEM((128, 16), jnp.float32),
            pltpu.VMEM((128, 16), jnp.float32),
            pltpu.SemaphoreType.DMA,
            pltpu.SemaphoreType.DMA,
            pltpu.SemaphoreType.DMA,
            pltpu.SemaphoreType.DMA,
            pltpu.SemaphoreType.DMA,
            pltpu.SemaphoreType.DMA,
        ],
    )(_sc_gather_prod_body)


def _mul_reduce(za, zb, ro):
    def mbody(r, _):
        acc = za[r, pl.ds(0, 16)] * zb[r, pl.ds(0, 16)]
        for k2 in range(1, 7):
            sl = pl.ds(k2 * 16, 16)
            acc = acc + za[r, sl] * zb[r, sl]
        ro[r, pl.ds(0, 16)] = acc
        return 0

    lax.fori_loop(0, 128, mbody, 0)


def _sc_gather_prod_body(z112, rowg2d, colg2d, n0g2d, n1g2d, outp, outn,
                         av, bv, za0, zb0, za1, zb1, ro0, ro1,
                         sa0, sb0, sa1, sb1, w0, w1):
    c = lax.axis_index("c")
    s = lax.axis_index("s")
    rbase = jnp.where(c == 0, s * C0R, C0TOT + s * C1R)
    nrows = jnp.where(c == 0, C0R, C1R)
    for (aidx, bidx, out) in ((rowg2d, colg2d, outp), (n0g2d, n1g2d, outn)):
        pltpu.sync_copy(aidx.at[pl.ds(rbase, C1R)], av)
        pltpu.sync_copy(bidx.at[pl.ds(rbase, C1R)], bv)
        pltpu.async_copy(z112.at[av.at[0]], za0, sa0)
        pltpu.async_copy(z112.at[bv.at[0]], zb0, sb0)

        def bd(i, _):
            pltpu.async_copy(z112.at[av.at[2 * i + 1]], za1, sa1)
            pltpu.async_copy(z112.at[bv.at[2 * i + 1]], zb1, sb1)
            pltpu.make_async_copy(z112.at[av.at[2 * i]], za0, sa0).wait()
            pltpu.make_async_copy(z112.at[bv.at[2 * i]], zb0, sb0).wait()

            @pl.when(i > 0)
            def _():
                pltpu.make_async_copy(ro0, out.at[pl.ds(0, 128)], w0).wait()

            _mul_reduce(za0, zb0, ro0)
            pltpu.async_copy(ro0, out.at[pl.ds((rbase + 2 * i) * 128, 128)], w0)
            nxt = jnp.minimum(2 * i + 2, nrows - 1)
            pltpu.async_copy(z112.at[av.at[nxt]], za0, sa0)
            pltpu.async_copy(z112.at[bv.at[nxt]], zb0, sb0)
            pltpu.make_async_copy(z112.at[av.at[2 * i + 1]], za1, sa1).wait()
            pltpu.make_async_copy(z112.at[bv.at[2 * i + 1]], zb1, sb1).wait()

            @pl.when(i > 0)
            def _():
                pltpu.make_async_copy(ro1, out.at[pl.ds(0, 128)], w1).wait()

            _mul_reduce(za1, zb1, ro1)
            pltpu.async_copy(ro1, out.at[pl.ds((rbase + 2 * i + 1) * 128, 128)], w1)
            return 0

        lax.fori_loop(0, nrows // 2, bd, 0)
        pltpu.make_async_copy(ro0, out.at[pl.ds(0, 128)], w0).wait()
        pltpu.make_async_copy(ro1, out.at[pl.ds(0, 128)], w1).wait()
        pltpu.make_async_copy(z112.at[av.at[0]], za0, sa0).wait()
        pltpu.make_async_copy(z112.at[bv.at[0]], zb0, sb0).wait()


def _bn_spec(*block):
    return pl.BlockSpec(block, lambda i: (0,) * (len(block) - 2) + (i, 0))


def _w_spec(*block):
    return pl.BlockSpec(block, lambda i: (0,) * len(block))


def _tca_body(degp, xp, wl1, bl1, wl2, bl2, dis, xs0, xs1, l1, l2):
    deg = (degp[0] + degp[1])[:, :1]
    d = jnp.where(deg > 0, 1.0 / jnp.sqrt(jnp.maximum(deg, 1e-12)), 0.0)
    dis[...] = d
    xs = d * xp[...]
    xs0[...] = xs[:, :32]
    xs1[...] = xs[:, 32:]
    l1[...] = jnp.maximum(xp[...] @ wl1[...] + bl1[...], 0.0)
    l2[...] = jnp.maximum(xp[...] @ wl2[...] + bl2[...], 0.0)


def _tcb_body(xp, dis, s1a, s1b, w10, w11, b1, w21, h, y0, y1, y2, y3):
    t = jnp.concatenate([s1a[0] + s1a[1], s1b[0] + s1b[1]], axis=1)
    tx1 = -dis[...] * t
    hv = jnp.maximum(xp[...] @ w10[...] + tx1 @ w11[...] + b1[...], 0.0)
    h[...] = hv
    y = dis[...] * (hv @ w21[...])
    y0[...] = y[:, 0:32]
    y1[...] = y[:, 32:64]
    y2[...] = y[:, 64:96]
    y3[...] = y[:, 96:128]


def _tcc_body(h, dis, l1, l2, s20, s21, s22, s23, w20, b2, w30, b3, w31r,
              z112, y3s, outp):
    t2 = jnp.concatenate([s20[0] + s20[1], s21[0] + s21[1],
                          s22[0] + s22[1], s23[0] + s23[1]], axis=1)
    x1 = jnp.maximum(h[...] @ w20[...] - dis[...] * t2 + b2[...], 0.0)
    xa = x1 + l1[...]
    z = x1 + l2[...]
    z112[...] = z[:, :112]
    y3s[...] = dis[...] * (xa @ w31r[...])
    outp[...] = xa @ w30[...] + b3[...]


def _tcd1_body(outp, dis, s3, o):
    o[...] = outp[...] - dis[...] * (s3[0][:, :1] + s3[1][:, :1])


def _tcd2_body(pp, pn, o):
    i = pl.program_id(0)
    e = i * BE + lax.broadcasted_iota(jnp.int32, (BE, 1), 0)
    valid = e < E
    lp = jnp.sum(pp[...], axis=1, keepdims=True)
    ln = jnp.sum(pn[...], axis=1, keepdims=True)
    tp = -jnp.log(jax.nn.sigmoid(lp) + 1e-15)
    tn = -jnp.log(1.0 - jax.nn.sigmoid(ln) + 1e-15)
    val = jnp.sum(jnp.where(valid, tp + tn, 0.0)) * (1.0 / E)

    @pl.when(i == 0)
    def _():
        o[0, 0] = 0.0

    o[0, 0] += val


def _pad2(a, r, c):
    return jnp.pad(a, ((0, r - a.shape[0]), (0, c - a.shape[1])))


def kernel(x, edge_index, W10, W11, b1, W20, W21, b2, W30, W31, b3,
           Wl1, bl1, Wl2, bl2, c1, c2):
    f32 = jnp.float32
    row, col = edge_index[0], edge_index[1]
    neg = jax.random.randint(jax.random.key(777), (2, E), 0, N, dtype=jnp.int32)

    pad0 = jnp.zeros((EP - E,), jnp.int32)
    padN = jnp.full((EP - E,), N, jnp.int32)
    rowg2d = jnp.concatenate([row, pad0]).reshape(R, 128)
    rows2d = jnp.concatenate([row, padN]).reshape(R, 128)
    cols2d = jnp.concatenate([col, padN]).reshape(R, 128)
    colg2d = jnp.concatenate([col, pad0]).reshape(R, 128)
    n0g2d = jnp.concatenate([neg[0], pad0]).reshape(R, 128)
    n1g2d = jnp.concatenate([neg[1], pad0]).reshape(R, 128)

    z16 = jnp.zeros((NACC, 16), f32)
    z32 = jnp.zeros((NACC, 32), f32)
    ones128 = jnp.ones((128, 16), f32)

    xp = _pad2(x, N, 64)
    w10 = _pad2(W10, 64, 384)
    w11 = _pad2(W11, 64, 384)
    b1p = _pad2(b1[None, :], 1, 384)
    w20 = _pad2(W20, 384, 128)
    w21 = _pad2(W21, 384, 128)
    b2p = _pad2(b2[None, :], 1, 128)
    w30 = _pad2(W30, 128, 1)
    b3p = b3[None, :]
    w31r = jnp.tile(_pad2(W31, 128, 1), (1, 16))
    wl1 = _pad2(Wl1, 64, 128)
    bl1p = _pad2(bl1[None, :], 1, 128)
    wl2 = _pad2(Wl2, 64, 128)
    bl2p = _pad2(bl2[None, :], 1, 128)

    grid_n = (N // BN,)

    degp = _make_degree()(rows2d, z16, ones128)

    dis, xs0, xs1, l1, l2 = pl.pallas_call(
        _tca_body,
        grid=grid_n,
        in_specs=[_bn_spec(2, BN, 16), _bn_spec(BN, 64),
                  _w_spec(64, 128), _w_spec(1, 128),
                  _w_spec(64, 128), _w_spec(1, 128)],
        out_specs=[_bn_spec(BN, 1), _bn_spec(BN, 32), _bn_spec(BN, 32),
                   _bn_spec(BN, 128), _bn_spec(BN, 128)],
        out_shape=[_f32(N, 1), _f32(N, 32), _f32(N, 32),
                   _f32(N, 128), _f32(N, 128)],
    )(degp, xp, wl1, bl1p, wl2, bl2p)

    s1a, s1b = _make_segsum(32, 2)(xs0, xs1, rowg2d, cols2d, z32)

    h, y0, y1, y2, y3 = pl.pallas_call(
        _tcb_body,
        grid=grid_n,
        in_specs=[_bn_spec(BN, 64), _bn_spec(BN, 1),
                  _bn_spec(2, BN, 32), _bn_spec(2, BN, 32),
                  _w_spec(64, 384), _w_spec(64, 384), _w_spec(1, 384),
                  _w_spec(384, 128)],
        out_specs=[_bn_spec(BN, 384)] + [_bn_spec(BN, 32)] * 4,
        out_shape=[_f32(N, 384)] + [_f32(N, 32)] * 4,
    )(xp, dis, s1a, s1b, w10, w11, b1p, w21)

    s2 = _make_segsum(32, 4)(y0, y1, y2, y3, rowg2d, cols2d, z32)

    z112, y3s, outp = pl.pallas_call(
        _tcc_body,
        grid=grid_n,
        in_specs=[_bn_spec(BN, 384), _bn_spec(BN, 1),
                  _bn_spec(BN, 128), _bn_spec(BN, 128)]
                 + [_bn_spec(2, BN, 32)] * 4
                 + [_w_spec(384, 128), _w_spec(1, 128), _w_spec(128, 1),
                    _w_spec(1, 1), _w_spec(128, 16)],
        out_specs=[_bn_spec(BN, 112), _bn_spec(BN, 16), _bn_spec(BN, 1)],
        out_shape=[_f32(N, 112), _f32(N, 16), _f32(N, 1)],
    )(h, dis, l1, l2, s2[0], s2[1], s2[2], s2[3], w20, b2p, w30, b3p, w31r)

    (s3,) = _make_segsum(16, 1)(y3s, rowg2d, cols2d, z16)

    prodp, prodn = _make_gather_prod()(z112, rowg2d, colg2d, n0g2d, n1g2d)

    out2d = pl.pallas_call(
        _tcd1_body,
        grid=grid_n,
        in_specs=[_bn_spec(BN, 1), _bn_spec(BN, 1), _bn_spec(2, BN, 16)],
        out_specs=_bn_spec(BN, 1),
        out_shape=_f32(N, 1),
    )(outp, dis, s3)

    loss2d = pl.pallas_call(
        _tcd2_body,
        grid=(EP // BE,),
        in_specs=[_bn_spec(BE, 16), _bn_spec(BE, 16)],
        out_specs=pl.BlockSpec((1, 1), lambda i: (0, 0), memory_space=pltpu.SMEM),
        out_shape=_f32(1, 1),
    )(prodp, prodn)

    out = out2d[:, 0]
    r_loss = loss2d[0, 0]
    return (out, r_loss, c1, c2)


# speedup vs baseline: 1.2200x; 1.2200x over previous
import functools

import jax
import jax.numpy as jnp
from jax import lax
from jax.experimental import pallas as pl
from jax.experimental.pallas import tpu as pltpu
from jax.experimental.pallas import tpu_sc as plsc

N = 50000
E = 800000
EP = 819200
R = EP // 128
RPW = R // 32
K = 8
KB = 50
C0R = 250
C1R = 2 * RPW - C0R
C0TOT = 16 * C0R
CMAX = max(C0R, C1R)
RIDX = R + CMAX
NACC = 50048
ZSTR = NACC // 16
OSTR = N // 16
BN = 2000
BE = 8192

def _f32(*shape):
    return jax.ShapeDtypeStruct(shape, jnp.float32)


@functools.lru_cache(maxsize=None)
def _mesh():
    return plsc.VectorSubcoreMesh(core_axis_name="c", subcore_axis_name="s")


@functools.lru_cache(maxsize=None)
def _make_degree():
    return functools.partial(
        pl.kernel,
        out_type=_f32(2, NACC, 16),
        mesh=_mesh(),
        compiler_params=pltpu.CompilerParams(use_tc_tiling_on_sc=False),
        scratch_types=[
            pltpu.VMEM((K, 128), jnp.int32),
            pltpu.VMEM((128, 16), jnp.float32),
            pltpu.VMEM_SHARED((NACC, 16), jnp.float32),
        ],
    )(_sc_degree_body)


def _sc_degree_body(rows2d, zeros16, ones128, out, rowv, onesv, acc):
    c = lax.axis_index("c")
    s = lax.axis_index("s")
    pltpu.sync_copy(zeros16.at[pl.ds(s * ZSTR, ZSTR)], acc.at[pl.ds(s * ZSTR, ZSTR)])
    pltpu.sync_copy(ones128, onesv)
    plsc.subcore_barrier()
    rbase = (c * 16 + s) * RPW

    def body(i, _):
        pltpu.sync_copy(rows2d.at[pl.ds(rbase + i * K, K)], rowv)
        for j in range(K):
            pltpu.sync_copy(onesv, acc.at[rowv.at[j]], add=True)
        return 0

    lax.fori_loop(0, RPW // K, body, 0)
    plsc.subcore_barrier()
    pltpu.sync_copy(acc.at[pl.ds(s * ZSTR, ZSTR)], out.at[c].at[pl.ds(s * ZSTR, ZSTR)])


@functools.lru_cache(maxsize=None)
def _make_segsum(D, nchunk):
    def body(*refs):
        tables = refs[:nchunk]
        rowg2d = refs[nchunk]
        cols2d = refs[nchunk + 1]
        zeros = refs[nchunk + 2]
        outs = refs[nchunk + 3:nchunk + 3 + nchunk]
        rowv, colv, g0, g1, acc, sa0, sa1, c0, c1 = refs[nchunk + 3 + nchunk:]
        c = lax.axis_index("c")
        s = lax.axis_index("s")
        rbase = jnp.where(c == 0, s * C0R, C0TOT + s * C1R)
        nblk = jnp.where(c == 0, C0R // KB, C1R // KB)
        for p in range(nchunk):
            pltpu.sync_copy(zeros.at[pl.ds(s * ZSTR, ZSTR)], acc.at[pl.ds(s * ZSTR, ZSTR)])
            plsc.subcore_barrier()

            def blk(b, _):
                pltpu.sync_copy(rowg2d.at[pl.ds(rbase + b * KB, KB)], rowv)
                pltpu.sync_copy(cols2d.at[pl.ds(rbase + b * KB, KB)], colv)
                pltpu.async_copy(tables[p].at[rowv.at[0]], g0, sa0)

                def bd(i, _):
                    pltpu.make_async_copy(tables[p].at[rowv.at[2 * i]], g0, sa0).wait()
                    pltpu.async_copy(g0, acc.at[colv.at[2 * i]], c0, add=True)

                    @pl.when(i > 0)
                    def _():
                        pltpu.make_async_copy(g1, acc.at[colv.at[0]], c1).wait()

                    pltpu.async_copy(tables[p].at[rowv.at[2 * i + 1]], g1, sa1)
                    pltpu.make_async_copy(tables[p].at[rowv.at[2 * i + 1]], g1, sa1).wait()
                    pltpu.async_copy(g1, acc.at[colv.at[2 * i + 1]], c1, add=True)
                    pltpu.make_async_copy(g0, acc.at[colv.at[0]], c0).wait()
                    nxt = jnp.minimum(2 * i + 2, KB - 1)
                    pltpu.async_copy(tables[p].at[rowv.at[nxt]], g0, sa0)
                    return 0

                lax.fori_loop(0, KB // 2, bd, 0)
                pltpu.make_async_copy(g1, acc.at[colv.at[0]], c1).wait()
                pltpu.make_async_copy(tables[p].at[rowv.at[0]], g0, sa0).wait()
                return 0

            lax.fori_loop(0, nblk, blk, 0)
            plsc.subcore_barrier()
            pltpu.sync_copy(acc.at[pl.ds(s * ZSTR, ZSTR)],
                            outs[p].at[c].at[pl.ds(s * ZSTR, ZSTR)])
            plsc.subcore_barrier()

    return functools.partial(
        pl.kernel,
        out_type=[_f32(2, NACC, D)] * nchunk,
        mesh=_mesh(),
        compiler_params=pltpu.CompilerParams(use_tc_tiling_on_sc=False),
        scratch_types=[
            pltpu.VMEM((KB, 128), jnp.int32),
            pltpu.VMEM((KB, 128), jnp.int32),
            pltpu.VMEM((128, D), jnp.float32),
            pltpu.VMEM((128, D), jnp.float32),
            pltpu.VMEM_SHARED((NACC, D), jnp.float32),
            pltpu.SemaphoreType.DMA,
            pltpu.SemaphoreType.DMA,
            pltpu.SemaphoreType.DMA,
            pltpu.SemaphoreType.DMA,
        ],
    )(body)


@functools.lru_cache(maxsize=None)
def _make_gather_prod():
    return functools.partial(
        pl.kernel,
        out_type=[_f32(EP, 16), _f32(EP, 16)],
        mesh=_mesh(),
        compiler_params=pltpu.CompilerParams(use_tc_tiling_on_sc=False),
        scratch_types=[
            pltpu.VMEM((CMAX, 128), jnp.int32),
            pltpu.VMEM((CMAX, 128), jnp.int32),
            pltpu.VMEM((128, 112), jnp.float32),
            pltpu.VMEM((128, 112), jnp.float32),
            pltpu.VMEM((128, 112), jnp.float32),
            pltpu.VMEM((128, 112), jnp.float32),
            pltpu.VMEM((128, 16), jnp.float32),
            pltpu.VMEM((128, 16), jnp.float32),
            pltpu.SemaphoreType.DMA,
            pltpu.SemaphoreType.DMA,
            pltpu.SemaphoreType.DMA,
            pltpu.SemaphoreType.DMA,
            pltpu.SemaphoreType.DMA,
            pltpu.SemaphoreType.DMA,
        ],
    )(_sc_gather_prod_body)


def _mul_reduce(za, zb, ro):
    def mbody(r, _):
        acc = za[r, pl.ds(0, 16)] * zb[r, pl.ds(0, 16)]
        for k2 in range(1, 7):
            sl = pl.ds(k2 * 16, 16)
            acc = acc + za[r, sl] * zb[r, sl]
        ro[r, pl.ds(0, 16)] = acc
        return 0

    lax.fori_loop(0, 128, mbody, 0)


def _sc_gather_prod_body(z112, rowg2d, colg2d, n0g2d, n1g2d, outp, outn,
                         av, bv, za0, zb0, za1, zb1, ro0, ro1,
                         sa0, sb0, sa1, sb1, w0, w1):
    c = lax.axis_index("c")
    s = lax.axis_index("s")
    rbase = jnp.where(c == 0, s * C0R, C0TOT + s * C1R)
    nrows = jnp.where(c == 0, C0R, C1R)
    for (aidx, bidx, out) in ((rowg2d, colg2d, outp), (n0g2d, n1g2d, outn)):
        pltpu.sync_copy(aidx.at[pl.ds(rbase, CMAX)], av)
        pltpu.sync_copy(bidx.at[pl.ds(rbase, CMAX)], bv)
        pltpu.async_copy(z112.at[av.at[0]], za0, sa0)
        pltpu.async_copy(z112.at[bv.at[0]], zb0, sb0)

        def bd(i, _):
            pltpu.async_copy(z112.at[av.at[2 * i + 1]], za1, sa1)
            pltpu.async_copy(z112.at[bv.at[2 * i + 1]], zb1, sb1)
            pltpu.make_async_copy(z112.at[av.at[2 * i]], za0, sa0).wait()
            pltpu.make_async_copy(z112.at[bv.at[2 * i]], zb0, sb0).wait()

            @pl.when(i > 0)
            def _():
                pltpu.make_async_copy(ro0, out.at[pl.ds(0, 128)], w0).wait()

            _mul_reduce(za0, zb0, ro0)
            pltpu.async_copy(ro0, out.at[pl.ds((rbase + 2 * i) * 128, 128)], w0)
            nxt = jnp.minimum(2 * i + 2, nrows - 1)
            pltpu.async_copy(z112.at[av.at[nxt]], za0, sa0)
            pltpu.async_copy(z112.at[bv.at[nxt]], zb0, sb0)
            pltpu.make_async_copy(z112.at[av.at[2 * i + 1]], za1, sa1).wait()
            pltpu.make_async_copy(z112.at[bv.at[2 * i + 1]], zb1, sb1).wait()

            @pl.when(i > 0)
            def _():
                pltpu.make_async_copy(ro1, out.at[pl.ds(0, 128)], w1).wait()

            _mul_reduce(za1, zb1, ro1)
            pltpu.async_copy(ro1, out.at[pl.ds((rbase + 2 * i + 1) * 128, 128)], w1)
            return 0

        lax.fori_loop(0, nrows // 2, bd, 0)
        pltpu.make_async_copy(ro0, out.at[pl.ds(0, 128)], w0).wait()
        pltpu.make_async_copy(ro1, out.at[pl.ds(0, 128)], w1).wait()
        pltpu.make_async_copy(z112.at[av.at[0]], za0, sa0).wait()
        pltpu.make_async_copy(z112.at[bv.at[0]], zb0, sb0).wait()


def _bn_spec(*block):
    return pl.BlockSpec(block, lambda i: (0,) * (len(block) - 2) + (i, 0))


def _w_spec(*block):
    return pl.BlockSpec(block, lambda i: (0,) * len(block))


def _tca_body(degp, xp, wl1, bl1, wl2, bl2, dis, xs0, xs1, l1, l2):
    deg = (degp[0] + degp[1])[:, :1]
    d = jnp.where(deg > 0, 1.0 / jnp.sqrt(jnp.maximum(deg, 1e-12)), 0.0)
    dis[...] = d
    xs = d * xp[...]
    xs0[...] = xs[:, :32]
    xs1[...] = xs[:, 32:]
    l1[...] = jnp.maximum(xp[...] @ wl1[...] + bl1[...], 0.0)
    l2[...] = jnp.maximum(xp[...] @ wl2[...] + bl2[...], 0.0)


def _tcb_body(xp, dis, s1a, s1b, w10, w11, b1, w21, h, y0, y1, y2, y3):
    t = jnp.concatenate([s1a[0] + s1a[1], s1b[0] + s1b[1]], axis=1)
    tx1 = -dis[...] * t
    hv = jnp.maximum(xp[...] @ w10[...] + tx1 @ w11[...] + b1[...], 0.0)
    h[...] = hv
    y = dis[...] * (hv @ w21[...])
    y0[...] = y[:, 0:32]
    y1[...] = y[:, 32:64]
    y2[...] = y[:, 64:96]
    y3[...] = y[:, 96:128]


def _tcc_body(h, dis, l1, l2, s20, s21, s22, s23, w20, b2, w30, b3, w31r,
              z112, y3s, outp):
    t2 = jnp.concatenate([s20[0] + s20[1], s21[0] + s21[1],
                          s22[0] + s22[1], s23[0] + s23[1]], axis=1)
    x1 = jnp.maximum(h[...] @ w20[...] - dis[...] * t2 + b2[...], 0.0)
    xa = x1 + l1[...]
    z = x1 + l2[...]
    z112[...] = z[:, :112]
    y3s[...] = dis[...] * (xa @ w31r[...])
    outp[...] = xa @ w30[...] + b3[...]


def _tcd1_body(outp, dis, s3, o):
    o[...] = outp[...] - dis[...] * (s3[0][:, :1] + s3[1][:, :1])


def _tcd2_body(pp, pn, o):
    i = pl.program_id(0)
    e = i * BE + lax.broadcasted_iota(jnp.int32, (BE, 1), 0)
    valid = e < E
    lp = jnp.sum(pp[...], axis=1, keepdims=True)
    ln = jnp.sum(pn[...], axis=1, keepdims=True)
    tp = -jnp.log(jax.nn.sigmoid(lp) + 1e-15)
    tn = -jnp.log(1.0 - jax.nn.sigmoid(ln) + 1e-15)
    val = jnp.sum(jnp.where(valid, tp + tn, 0.0)) * (1.0 / E)

    @pl.when(i == 0)
    def _():
        o[0, 0] = 0.0

    o[0, 0] += val


def _pad2(a, r, c):
    return jnp.pad(a, ((0, r - a.shape[0]), (0, c - a.shape[1])))


def kernel(x, edge_index, W10, W11, b1, W20, W21, b2, W30, W31, b3,
           Wl1, bl1, Wl2, bl2, c1, c2):
    f32 = jnp.float32
    row, col = edge_index[0], edge_index[1]
    neg = jax.random.randint(jax.random.key(777), (2, E), 0, N, dtype=jnp.int32)

    epad = RIDX * 128 - E
    pad0 = jnp.zeros((epad,), jnp.int32)
    padN = jnp.full((epad,), N, jnp.int32)
    rowg2d = jnp.concatenate([row, pad0]).reshape(RIDX, 128)
    rows2d = jnp.concatenate([row, padN]).reshape(RIDX, 128)
    cols2d = jnp.concatenate([col, padN]).reshape(RIDX, 128)
    colg2d = jnp.concatenate([col, pad0]).reshape(RIDX, 128)
    n0g2d = jnp.concatenate([neg[0], pad0]).reshape(RIDX, 128)
    n1g2d = jnp.concatenate([neg[1], pad0]).reshape(RIDX, 128)

    z16 = jnp.zeros((NACC, 16), f32)
    z32 = jnp.zeros((NACC, 32), f32)
    ones128 = jnp.ones((128, 16), f32)

    xp = _pad2(x, N, 64)
    w10 = _pad2(W10, 64, 384)
    w11 = _pad2(W11, 64, 384)
    b1p = _pad2(b1[None, :], 1, 384)
    w20 = _pad2(W20, 384, 128)
    w21 = _pad2(W21, 384, 128)
    b2p = _pad2(b2[None, :], 1, 128)
    w30 = _pad2(W30, 128, 1)
    b3p = b3[None, :]
    w31r = jnp.tile(_pad2(W31, 128, 1), (1, 16))
    wl1 = _pad2(Wl1, 64, 128)
    bl1p = _pad2(bl1[None, :], 1, 128)
    wl2 = _pad2(Wl2, 64, 128)
    bl2p = _pad2(bl2[None, :], 1, 128)

    grid_n = (N // BN,)

    degp = _make_degree()(rows2d, z16, ones128)

    dis, xs0, xs1, l1, l2 = pl.pallas_call(
        _tca_body,
        grid=grid_n,
        in_specs=[_bn_spec(2, BN, 16), _bn_spec(BN, 64),
                  _w_spec(64, 128), _w_spec(1, 128),
                  _w_spec(64, 128), _w_spec(1, 128)],
        out_specs=[_bn_spec(BN, 1), _bn_spec(BN, 32), _bn_spec(BN, 32),
                   _bn_spec(BN, 128), _bn_spec(BN, 128)],
        out_shape=[_f32(N, 1), _f32(N, 32), _f32(N, 32),
                   _f32(N, 128), _f32(N, 128)],
    )(degp, xp, wl1, bl1p, wl2, bl2p)

    s1a, s1b = _make_segsum(32, 2)(xs0, xs1, rowg2d, cols2d, z32)

    h, y0, y1, y2, y3 = pl.pallas_call(
        _tcb_body,
        grid=grid_n,
        in_specs=[_bn_spec(BN, 64), _bn_spec(BN, 1),
                  _bn_spec(2, BN, 32), _bn_spec(2, BN, 32),
                  _w_spec(64, 384), _w_spec(64, 384), _w_spec(1, 384),
                  _w_spec(384, 128)],
        out_specs=[_bn_spec(BN, 384)] + [_bn_spec(BN, 32)] * 4,
        out_shape=[_f32(N, 384)] + [_f32(N, 32)] * 4,
    )(xp, dis, s1a, s1b, w10, w11, b1p, w21)

    s2 = _make_segsum(32, 4)(y0, y1, y2, y3, rowg2d, cols2d, z32)

    z112, y3s, outp = pl.pallas_call(
        _tcc_body,
        grid=grid_n,
        in_specs=[_bn_spec(BN, 384), _bn_spec(BN, 1),
                  _bn_spec(BN, 128), _bn_spec(BN, 128)]
                 + [_bn_spec(2, BN, 32)] * 4
                 + [_w_spec(384, 128), _w_spec(1, 128), _w_spec(128, 1),
                    _w_spec(1, 1), _w_spec(128, 16)],
        out_specs=[_bn_spec(BN, 112), _bn_spec(BN, 16), _bn_spec(BN, 1)],
        out_shape=[_f32(N, 112), _f32(N, 16), _f32(N, 1)],
    )(h, dis, l1, l2, s2[0], s2[1], s2[2], s2[3], w20, b2p, w30, b3p, w31r)

    (s3,) = _make_segsum(16, 1)(y3s, rowg2d, cols2d, z16)

    prodp, prodn = _make_gather_prod()(z112, rowg2d, colg2d, n0g2d, n1g2d)

    out2d = pl.pallas_call(
        _tcd1_body,
        grid=grid_n,
        in_specs=[_bn_spec(BN, 1), _bn_spec(BN, 1), _bn_spec(2, BN, 16)],
        out_specs=_bn_spec(BN, 1),
        out_shape=_f32(N, 1),
    )(outp, dis, s3)

    loss2d = pl.pallas_call(
        _tcd2_body,
        grid=(EP // BE,),
        in_specs=[_bn_spec(BE, 16), _bn_spec(BE, 16)],
        out_specs=pl.BlockSpec((1, 1), lambda i: (0, 0), memory_space=pltpu.SMEM),
        out_shape=_f32(1, 1),
    )(prodp, prodn)

    out = out2d[:, 0]
    r_loss = loss2d[0, 0]
    return (out, r_loss, c1, c2)
